# baseline (speedup 1.0000x reference)
import functools

import jax
import jax.numpy as jnp
from jax import lax
from jax.experimental import pallas as pl
from jax.experimental.pallas import tpu as pltpu
from jax.experimental.pallas import tpu_sc as plsc

N_IDX = 100000
D = 128
C = 128
NW = 32
K = 5
NB = 5
CPW = K * NB
N_CHUNKS = NW * CPW
B_PAD = N_CHUNKS * C

_mesh = plsc.VectorSubcoreMesh(core_axis_name="c", subcore_axis_name="s")


@functools.partial(
    pl.kernel,
    mesh=_mesh,
    out_type=jax.ShapeDtypeStruct((B_PAD, D), jnp.float32),
    scratch_types=(
        [pltpu.VMEM((C,), jnp.int32) for _ in range(K)]
        + [pltpu.VMEM((K, C, D), jnp.float32),
           pltpu.SemaphoreType.DMA,
           pltpu.SemaphoreType.DMA]
    ),
)
def _sc_gather(x_hbm, idx_hbm, out_hbm, *rest):
    idx_bufs, (rows_v, gsem, osem) = rest[:K], rest[K:]
    w = lax.axis_index("s") * 2 + lax.axis_index("c")

    def body(s, carry):
        for b in range(K):
            pltpu.sync_copy(idx_hbm.at[(s * K + b) * NW + w], idx_bufs[b])
        gds = [
            pltpu.async_copy(x_hbm.at[idx_bufs[b]], rows_v.at[b], gsem)
            for b in range(K)
        ]
        for gd in gds:
            gd.wait()
        ods = [
            pltpu.async_copy(rows_v.at[b],
                             out_hbm.at[pl.ds(((s * K + b) * NW + w) * C, C)],
                             osem)
            for b in range(K)
        ]
        for od in ods:
            od.wait()
        return carry

    lax.fori_loop(0, NB, body, 0)


def kernel(x, idx):
    idx32 = idx.astype(jnp.int32)
    idx_pad = jnp.zeros((B_PAD,), jnp.int32).at[:N_IDX].set(idx32)
    out = _sc_gather(x, idx_pad.reshape(N_CHUNKS, C))
    return out[:N_IDX]


# speedup vs baseline: 1.0659x; 1.0659x over previous
import functools

import jax
import jax.numpy as jnp
from jax import lax
from jax.experimental import pallas as pl
from jax.experimental.pallas import tpu as pltpu
from jax.experimental.pallas import tpu_sc as plsc

N_IDX = 100000
D = 128
C = 128
NW = 32
CPW = 25
N_CHUNKS = NW * CPW
B_PAD = N_CHUNKS * C

_mesh = plsc.VectorSubcoreMesh(core_axis_name="c", subcore_axis_name="s")


@functools.partial(
    pl.kernel,
    mesh=_mesh,
    out_type=jax.ShapeDtypeStruct((N_CHUNKS, C, D), jnp.float32),
    scratch_types=[
        pltpu.VMEM((C,), jnp.int32),
        pltpu.VMEM((C,), jnp.int32),
        pltpu.VMEM((2, C, D), jnp.float32),
        pltpu.SemaphoreType.DMA,
        pltpu.SemaphoreType.DMA,
    ],
)
def _sc_gather(x_hbm, idx_hbm, out_hbm, idx_a, idx_b, rows_v, gsem, osem):
    w = lax.axis_index("s") * 2 + lax.axis_index("c")
    idx_bufs = (idx_a, idx_b)

    def stage_and_gather(j):
        buf = j % 2
        pltpu.sync_copy(idx_hbm.at[j * NW + w], idx_bufs[buf])
        return pltpu.async_copy(x_hbm.at[idx_bufs[buf]], rows_v.at[buf], gsem)

    def start_out(j):
        return pltpu.async_copy(
            rows_v.at[j % 2], out_hbm.at[j * NW + w], osem)

    gds = [None] * CPW
    ods = [None] * CPW
    gds[0] = stage_and_gather(0)
    for j in range(CPW):
        if j >= 1:
            ods[j - 1].wait()
        if j + 1 < CPW:
            gds[j + 1] = stage_and_gather(j + 1)
        gds[j].wait()
        ods[j] = start_out(j)
    ods[CPW - 1].wait()


def kernel(x, idx):
    idx32 = idx.astype(jnp.int32)
    idx_pad = jnp.zeros((B_PAD,), jnp.int32).at[:N_IDX].set(idx32)
    out = _sc_gather(x, idx_pad.reshape(N_CHUNKS, C))
    return out.reshape(B_PAD, D)[:N_IDX]


# speedup vs baseline: 1.6432x; 1.5416x over previous
import functools

import jax
import jax.numpy as jnp
from jax import lax
from jax.experimental import pallas as pl
from jax.experimental.pallas import tpu as pltpu
from jax.experimental.pallas import tpu_sc as plsc

N_IDX = 100000
D = 128
C = 128
NW = 32
N_CHUNKS = -(-N_IDX // C)
B_PAD = N_CHUNKS * C
MAX_CHUNKS_PER_W = -(-N_CHUNKS // NW)

_mesh = plsc.VectorSubcoreMesh(core_axis_name="c", subcore_axis_name="s")


@functools.partial(
    pl.kernel,
    mesh=_mesh,
    out_type=jax.ShapeDtypeStruct((B_PAD, D), jnp.float32),
    scratch_types=[
        pltpu.VMEM((C,), jnp.int32),
        pltpu.VMEM((C, D), jnp.float32),
        pltpu.SemaphoreType.DMA,
    ],
)
def _sc_gather(x_hbm, idx_hbm, out_hbm, idx_v, rows_v, sem):
    w = lax.axis_index("s") * 2 + lax.axis_index("c")

    def body(j, carry):
        g = j * NW + w

        @pl.when(g < N_CHUNKS)
        def _():
            pltpu.sync_copy(idx_hbm.at[g], idx_v)
            pltpu.async_copy(x_hbm.at[idx_v], rows_v, sem).wait()
            pltpu.sync_copy(rows_v, out_hbm.at[pl.ds(g * C, C)])

        return carry

    lax.fori_loop(0, MAX_CHUNKS_PER_W, body, 0)


def kernel(x, idx):
    idx32 = idx.astype(jnp.int32)
    idx_pad = jnp.zeros((B_PAD,), jnp.int32).at[:N_IDX].set(idx32)
    out = _sc_gather(x, idx_pad.reshape(N_CHUNKS, C))
    return out[:N_IDX]


# speedup vs baseline: 2.3226x; 1.4135x over previous
import functools

import jax
import jax.numpy as jnp
from jax import lax
from jax.experimental import pallas as pl
from jax.experimental.pallas import tpu as pltpu
from jax.experimental.pallas import tpu_sc as plsc

N_IDX = 100000
D = 128
C = 128
NW = 32
N_FULL = N_IDX // C
C_TAIL = N_IDX - N_FULL * C
N_CHUNKS = N_FULL + 1
B_PAD = N_CHUNKS * C
MAX_CHUNKS_PER_W = -(-N_CHUNKS // NW)

_mesh = plsc.VectorSubcoreMesh(core_axis_name="c", subcore_axis_name="s")


@functools.partial(
    pl.kernel,
    mesh=_mesh,
    out_type=jax.ShapeDtypeStruct((N_IDX, D), jnp.float32),
    scratch_types=[
        pltpu.VMEM((C,), jnp.int32),
        pltpu.VMEM((C, D), jnp.float32),
        pltpu.SemaphoreType.DMA,
    ],
)
def _sc_gather(x_hbm, idx_hbm, out_hbm, idx_v, rows_v, sem):
    w = lax.axis_index("s") * 2 + lax.axis_index("c")

    def body(j, carry):
        g = j * NW + w

        @pl.when(g < N_FULL)
        def _():
            pltpu.sync_copy(idx_hbm.at[g], idx_v)
            pltpu.async_copy(x_hbm.at[idx_v], rows_v, sem).wait()
            pltpu.sync_copy(rows_v, out_hbm.at[pl.ds(g * C, C)])

        @pl.when(g == N_FULL)
        def _():
            pltpu.sync_copy(idx_hbm.at[g], idx_v)
            pltpu.async_copy(x_hbm.at[idx_v.at[pl.ds(0, C_TAIL)]],
                             rows_v.at[pl.ds(0, C_TAIL)], sem).wait()
            pltpu.sync_copy(rows_v.at[pl.ds(0, C_TAIL)],
                            out_hbm.at[pl.ds(g * C, C_TAIL)])

        return carry

    lax.fori_loop(0, MAX_CHUNKS_PER_W, body, 0)


def kernel(x, idx):
    idx32 = idx.astype(jnp.int32)
    idx_pad = jnp.zeros((B_PAD,), jnp.int32).at[:N_IDX].set(idx32)
    return _sc_gather(x, idx_pad.reshape(N_CHUNKS, C))
